# baseline (speedup 1.0000x reference)
import functools

import jax
import jax.numpy as jnp
from jax import lax
from jax.experimental import pallas as pl
from jax.experimental.pallas import tpu as pltpu
from jax.experimental.pallas import tpu_sc as plsc

_N_ATOMS = 100000
_FDIM = 128
_HIDDEN = 300
_N_GROUPS = 20000
_GSIZE = 8
_N_FUNC = 64

_NW = 32
_K = 128
_NCHUNK = 40
_EPW = _K * _NCHUNK
_E_PAD = _NW * _EPW
_NBUF = 4
_MROWS = 160
_SLAB_ROWS = 128
_RPT = _N_FUNC // 16


def _sc_body(f_hbm, a_hbm, b_hbm, z_hbm, out_hbm,
             aidx_v, bidx_v, rows_v, acc_v, tmp_v, slabs_sh, gsems):
    cid = lax.axis_index("c")
    sid = lax.axis_index("s")
    wid = sid * 2 + cid

    pltpu.sync_copy(a_hbm.at[wid], aidx_v)
    pltpu.sync_copy(b_hbm.at[wid], bidx_v)

    pltpu.async_copy(f_hbm.at[aidx_v.at[0]], rows_v.at[0], gsems.at[0])
    pltpu.async_copy(f_hbm.at[aidx_v.at[1]], rows_v.at[1], gsems.at[1])
    pltpu.sync_copy(z_hbm, slabs_sh.at[sid])

    @pl.loop(0, _NCHUNK, step=_NBUF)
    def _(j):
        for b in range(_NBUF):
            i = j + b
            pltpu.make_async_copy(f_hbm.at[aidx_v.at[i]], rows_v.at[b],
                                  gsems.at[b]).wait()
            nb = (b + 2) % _NBUF

            @pl.when(i + 2 < _NCHUNK)
            def _():
                pltpu.async_copy(f_hbm.at[aidx_v.at[i + 2]], rows_v.at[nb],
                                 gsems.at[nb])

            pltpu.sync_copy(rows_v.at[b], slabs_sh.at[sid].at[bidx_v.at[i]],
                            add=True)

    plsc.subcore_barrier()

    r0 = sid * _RPT
    pltpu.sync_copy(slabs_sh.at[0, pl.ds(r0, _RPT)], acc_v)
    for s in range(1, 16):
        pltpu.sync_copy(slabs_sh.at[s, pl.ds(r0, _RPT)], tmp_v)
        for r in range(_RPT):
            for c in range(_FDIM // 16):
                sl = pl.ds(c * 16, 16)
                acc_v[r, sl] = acc_v[r, sl] + tmp_v[r, sl]
    pltpu.sync_copy(acc_v, out_hbm.at[cid, pl.ds(r0, _RPT)])


@functools.cache
def _sc_edge_sum():
    return functools.partial(
        pl.kernel,
        out_type=jax.ShapeDtypeStruct((2, _N_FUNC, _FDIM), jnp.float32),
        mesh=plsc.VectorSubcoreMesh(core_axis_name="c", subcore_axis_name="s"),
        scratch_types=[
            pltpu.VMEM((_NCHUNK, _K), jnp.int32),
            pltpu.VMEM((_NCHUNK, _K), jnp.int32),
            pltpu.VMEM((_NBUF, _K, _FDIM), jnp.float32),
            pltpu.VMEM((_RPT, _FDIM), jnp.float32),
            pltpu.VMEM((_RPT, _FDIM), jnp.float32),
            pltpu.VMEM_SHARED((16, _SLAB_ROWS, _FDIM), jnp.float32),
            pltpu.SemaphoreType.DMA((_NBUF,)),
        ],
    )(_sc_body)


def _tc_body(s2_ref, w_ref, fs_ref, m_ref, out_ref, cnt_ref):
    s = s2_ref[0] + s2_ref[1]
    out_ref[...] = fs_ref[...] + jnp.dot(
        s, w_ref[...], preferred_element_type=jnp.float32)

    iota = lax.broadcasted_iota(jnp.int32, (_N_FUNC, 128), 0)

    def body(r, acc):
        blk = m_ref[pl.ds(r, 1), :]
        return acc + (jnp.broadcast_to(blk, (_N_FUNC, 128)) == iota
                      ).astype(jnp.int32)

    acc = lax.fori_loop(0, _MROWS, body,
                        jnp.zeros((_N_FUNC, 128), jnp.int32))
    cnt_ref[...] = jnp.sum(acc, axis=1, keepdims=True) + 1


def kernel(step, f_atoms, func2atom, mapping, W_i_atom, func_save):
    del step
    a = func2atom.reshape(-1).astype(jnp.int32)
    b = jnp.broadcast_to(mapping.astype(jnp.int32)[:, None],
                         (_N_GROUPS, _GSIZE)).reshape(-1)
    b = jnp.where(a == 0, _N_FUNC, b)
    a = jnp.maximum(a - 1, 0)
    pad = _E_PAD - a.shape[0]
    a = jnp.concatenate([a, jnp.zeros((pad,), jnp.int32)])
    b = jnp.concatenate([b, jnp.full((pad,), _N_FUNC, jnp.int32)])
    a3 = a.reshape(_NW, _NCHUNK, _K)
    b3 = b.reshape(_NW, _NCHUNK, _K)
    zeros = jnp.zeros((_SLAB_ROWS, _FDIM), jnp.float32)

    s2 = _sc_edge_sum()(f_atoms, a3, b3, zeros)

    mp = jnp.concatenate([
        mapping.astype(jnp.int32),
        jnp.full((_MROWS * 128 - _N_GROUPS,), -1, jnp.int32),
    ]).reshape(_MROWS, 128)

    func_save_new, cnt = pl.pallas_call(
        _tc_body,
        out_shape=(
            jax.ShapeDtypeStruct((_N_FUNC, _HIDDEN), jnp.float32),
            jax.ShapeDtypeStruct((_N_FUNC, 1), jnp.int32),
        ),
    )(s2, W_i_atom, func_save, mp)

    return func_save_new, cnt.reshape(_N_FUNC)


# speedup vs baseline: 1.3369x; 1.3369x over previous
import functools

import jax
import jax.numpy as jnp
from jax import lax
from jax.experimental import pallas as pl
from jax.experimental.pallas import tpu as pltpu
from jax.experimental.pallas import tpu_sc as plsc

_N_ATOMS = 100000
_FDIM = 128
_HIDDEN = 300
_N_GROUPS = 20000
_GSIZE = 8
_N_FUNC = 64

_K = 128
_N0 = 72
_N1 = 8
_TOT_CHUNK = 16 * (_N0 + _N1)
_CH_PAD = _TOT_CHUNK + _N0 - _N1
_E_PAD = _CH_PAD * _K
_NBUF = 4
_MROWS = 160
_SLAB_ROWS = 128
_RPT = _N_FUNC // 16


def _sc_body(f_hbm, a_hbm, b_hbm, out_hbm,
             aidx_v, bidx_v, rows_v, acc_v, tmp_v, zv, slabs_sh, gsems):
    cid = lax.axis_index("c")
    sid = lax.axis_index("s")
    base = jnp.where(cid == 0, sid * _N0, 16 * _N0 + sid * _N1)
    nch = jnp.where(cid == 0, _N0, _N1)

    @pl.when(cid == 0)
    def _():
        pltpu.sync_copy(a_hbm.at[pl.ds(base, _N0)], aidx_v)
        pltpu.sync_copy(b_hbm.at[pl.ds(base, _N0)], bidx_v)

    @pl.when(cid == 1)
    def _():
        pltpu.sync_copy(a_hbm.at[pl.ds(base, _N1)], aidx_v.at[pl.ds(0, _N1)])
        pltpu.sync_copy(b_hbm.at[pl.ds(base, _N1)], bidx_v.at[pl.ds(0, _N1)])

    pltpu.async_copy(f_hbm.at[aidx_v.at[0]], rows_v.at[0], gsems.at[0])
    pltpu.async_copy(f_hbm.at[aidx_v.at[1]], rows_v.at[1], gsems.at[1])

    zvec = jnp.zeros((16,), jnp.float32)
    for r in range(_N_FUNC + 1):
        for c in range(_FDIM // 16):
            zv[r, pl.ds(c * 16, 16)] = zvec
    pltpu.sync_copy(zv, slabs_sh.at[sid, pl.ds(0, _N_FUNC + 1)])

    @pl.loop(0, nch, step=_NBUF)
    def _(j):
        for b in range(_NBUF):
            i = j + b
            pltpu.make_async_copy(f_hbm.at[aidx_v.at[i]], rows_v.at[b],
                                  gsems.at[b]).wait()
            nb = (b + 2) % _NBUF

            @pl.when(i + 2 < nch)
            def _():
                pltpu.async_copy(f_hbm.at[aidx_v.at[i + 2]], rows_v.at[nb],
                                 gsems.at[nb])

            pltpu.sync_copy(rows_v.at[b], slabs_sh.at[sid].at[bidx_v.at[i]],
                            add=True)

    plsc.subcore_barrier()

    r0 = sid * _RPT
    pltpu.sync_copy(slabs_sh.at[0, pl.ds(r0, _RPT)], acc_v)
    for s in range(1, 16):
        pltpu.sync_copy(slabs_sh.at[s, pl.ds(r0, _RPT)], tmp_v)
        for r in range(_RPT):
            for c in range(_FDIM // 16):
                sl = pl.ds(c * 16, 16)
                acc_v[r, sl] = acc_v[r, sl] + tmp_v[r, sl]
    pltpu.sync_copy(acc_v, out_hbm.at[cid, pl.ds(r0, _RPT)])


@functools.cache
def _sc_edge_sum():
    return functools.partial(
        pl.kernel,
        out_type=jax.ShapeDtypeStruct((2, _N_FUNC, _FDIM), jnp.float32),
        mesh=plsc.VectorSubcoreMesh(core_axis_name="c", subcore_axis_name="s"),
        scratch_types=[
            pltpu.VMEM((_N0, _K), jnp.int32),
            pltpu.VMEM((_N0, _K), jnp.int32),
            pltpu.VMEM((_NBUF, _K, _FDIM), jnp.float32),
            pltpu.VMEM((_RPT, _FDIM), jnp.float32),
            pltpu.VMEM((_RPT, _FDIM), jnp.float32),
            pltpu.VMEM((_N_FUNC + 1, _FDIM), jnp.float32),
            pltpu.VMEM_SHARED((16, _SLAB_ROWS, _FDIM), jnp.float32),
            pltpu.SemaphoreType.DMA((_NBUF,)),
        ],
    )(_sc_body)


def _tc_body(s2_ref, w_ref, fs_ref, m_ref, out_ref, cnt_ref):
    s = s2_ref[0] + s2_ref[1]
    out_ref[...] = fs_ref[...] + jnp.dot(
        s, w_ref[...], preferred_element_type=jnp.float32)

    iota = lax.broadcasted_iota(jnp.int32, (_N_FUNC, 128), 0)

    def body(r, acc):
        blk = m_ref[pl.ds(r, 1), :]
        return acc + (jnp.broadcast_to(blk, (_N_FUNC, 128)) == iota
                      ).astype(jnp.int32)

    acc = lax.fori_loop(0, _MROWS, body,
                        jnp.zeros((_N_FUNC, 128), jnp.int32))
    cnt_ref[...] = jnp.sum(acc, axis=1, keepdims=True) + 1


def kernel(step, f_atoms, func2atom, mapping, W_i_atom, func_save):
    del step
    a = func2atom.reshape(-1).astype(jnp.int32)
    b = jnp.broadcast_to(mapping.astype(jnp.int32)[:, None],
                         (_N_GROUPS, _GSIZE)).reshape(-1)
    b = jnp.where(a == 0, _N_FUNC, b)
    a = jnp.maximum(a - 1, 0)
    pad = _E_PAD - a.shape[0]
    a = jnp.concatenate([a, jnp.zeros((pad,), jnp.int32)])
    b = jnp.concatenate([b, jnp.full((pad,), _N_FUNC, jnp.int32)])
    a3 = a.reshape(_CH_PAD, _K)
    b3 = b.reshape(_CH_PAD, _K)

    s2 = _sc_edge_sum()(f_atoms, a3, b3)

    mp = jnp.concatenate([
        mapping.astype(jnp.int32),
        jnp.full((_MROWS * 128 - _N_GROUPS,), -1, jnp.int32),
    ]).reshape(_MROWS, 128)

    func_save_new, cnt = pl.pallas_call(
        _tc_body,
        out_shape=(
            jax.ShapeDtypeStruct((_N_FUNC, _HIDDEN), jnp.float32),
            jax.ShapeDtypeStruct((_N_FUNC, 1), jnp.int32),
        ),
    )(s2, W_i_atom, func_save, mp)

    return func_save_new, cnt.reshape(_N_FUNC)


# speedup vs baseline: 1.3762x; 1.0294x over previous
import functools

import jax
import jax.numpy as jnp
from jax import lax
from jax.experimental import pallas as pl
from jax.experimental.pallas import tpu as pltpu
from jax.experimental.pallas import tpu_sc as plsc

_N_ATOMS = 100000
_FDIM = 128
_HIDDEN = 300
_N_GROUPS = 20000
_GSIZE = 8
_N_FUNC = 64

_K = 128
_N0 = 72
_N1 = 8
_TOT_CHUNK = 16 * (_N0 + _N1)
_CH_PAD = _TOT_CHUNK + _N0 - _N1
_E_PAD = _CH_PAD * _K
_NBUF = 4
_MROWS = 160
_SLAB_ROWS = 128
_RPT = _N_FUNC // 16


def _sc_body(f_hbm, a_hbm, b_hbm, out_hbm,
             aidx_v, bidx_v, rows_v, acc_v, tmp_v, zv, slabs_sh, gsems):
    cid = lax.axis_index("c")
    sid = lax.axis_index("s")
    base = jnp.where(cid == 0, sid * _N0, 16 * _N0 + sid * _N1)
    nch = jnp.where(cid == 0, _N0, _N1)

    @pl.when(cid == 0)
    def _():
        pltpu.sync_copy(a_hbm.at[pl.ds(base, _N0)], aidx_v)
        pltpu.sync_copy(b_hbm.at[pl.ds(base, _N0)], bidx_v)

    @pl.when(cid == 1)
    def _():
        pltpu.sync_copy(a_hbm.at[pl.ds(base, _N1)], aidx_v.at[pl.ds(0, _N1)])
        pltpu.sync_copy(b_hbm.at[pl.ds(base, _N1)], bidx_v.at[pl.ds(0, _N1)])

    pltpu.async_copy(f_hbm.at[aidx_v.at[0]], rows_v.at[0], gsems.at[0])
    pltpu.async_copy(f_hbm.at[aidx_v.at[1]], rows_v.at[1], gsems.at[1])

    zvec = jnp.zeros((16,), jnp.float32)
    for r in range(_N_FUNC + 1):
        for c in range(_FDIM // 16):
            zv[r, pl.ds(c * 16, 16)] = zvec
    pltpu.sync_copy(zv, slabs_sh.at[sid, pl.ds(0, _N_FUNC + 1)])

    @pl.loop(0, nch, step=_NBUF)
    def _(j):
        for b in range(_NBUF):
            i = j + b
            pltpu.make_async_copy(f_hbm.at[aidx_v.at[i]], rows_v.at[b],
                                  gsems.at[b]).wait()
            nb = (b + 2) % _NBUF

            @pl.when(i + 2 < nch)
            def _():
                pltpu.async_copy(f_hbm.at[aidx_v.at[i + 2]], rows_v.at[nb],
                                 gsems.at[nb])

            pltpu.sync_copy(rows_v.at[b], slabs_sh.at[sid].at[bidx_v.at[i]],
                            add=True)

    plsc.subcore_barrier()

    r0 = sid * _RPT
    pltpu.sync_copy(slabs_sh.at[0, pl.ds(r0, _RPT)], acc_v)
    for s in range(1, 16):
        pltpu.sync_copy(slabs_sh.at[s, pl.ds(r0, _RPT)], tmp_v)
        for r in range(_RPT):
            for c in range(_FDIM // 16):
                sl = pl.ds(c * 16, 16)
                acc_v[r, sl] = acc_v[r, sl] + tmp_v[r, sl]
    pltpu.sync_copy(acc_v, out_hbm.at[cid, pl.ds(r0, _RPT)])


@functools.cache
def _sc_edge_sum():
    return functools.partial(
        pl.kernel,
        out_type=jax.ShapeDtypeStruct((2, _N_FUNC, _FDIM), jnp.float32),
        mesh=plsc.VectorSubcoreMesh(core_axis_name="c", subcore_axis_name="s"),
        scratch_types=[
            pltpu.VMEM((_N0, _K), jnp.int32),
            pltpu.VMEM((_N0, _K), jnp.int32),
            pltpu.VMEM((_NBUF, _K, _FDIM), jnp.float32),
            pltpu.VMEM((_RPT, _FDIM), jnp.float32),
            pltpu.VMEM((_RPT, _FDIM), jnp.float32),
            pltpu.VMEM((_N_FUNC + 1, _FDIM), jnp.float32),
            pltpu.VMEM_SHARED((16, _SLAB_ROWS, _FDIM), jnp.float32),
            pltpu.SemaphoreType.DMA((_NBUF,)),
        ],
    )(_sc_body)


_EROWS = _N_GROUPS * _GSIZE // _K


def _prep_body(f2a_ref, map_ref, a_ref, b_ref):
    a_blk = f2a_ref[...]
    parts = [jnp.broadcast_to(map_ref[:, pl.ds(g, 1)], (_EROWS, _GSIZE))
             for g in range(16)]
    b_blk = jnp.concatenate(parts, axis=1)
    b_blk = jnp.where(a_blk == 0, _N_FUNC, b_blk)
    a_blk = jnp.maximum(a_blk - 1, 0)
    npad = _CH_PAD - _EROWS
    a_ref[...] = jnp.concatenate(
        [a_blk, jnp.zeros((npad, _K), jnp.int32)], axis=0)
    b_ref[...] = jnp.concatenate(
        [b_blk, jnp.full((npad, _K), _N_FUNC, jnp.int32)], axis=0)


def _tc_body(s2_ref, w_ref, fs_ref, m_ref, out_ref, cnt_ref):
    s = s2_ref[0] + s2_ref[1]
    out_ref[...] = fs_ref[...] + jnp.dot(
        s, w_ref[...], preferred_element_type=jnp.float32)

    iota = lax.broadcasted_iota(jnp.int32, (_N_FUNC, 128), 0)

    def body(r, acc):
        blk = m_ref[pl.ds(r, 1), :]
        return acc + (jnp.broadcast_to(blk, (_N_FUNC, 128)) == iota
                      ).astype(jnp.int32)

    acc = lax.fori_loop(0, _MROWS, body,
                        jnp.zeros((_N_FUNC, 128), jnp.int32))
    cnt_ref[...] = jnp.sum(acc, axis=1, keepdims=True) + 1


def kernel(step, f_atoms, func2atom, mapping, W_i_atom, func_save):
    del step
    mapping = mapping.astype(jnp.int32)
    f2a2 = func2atom.astype(jnp.int32).reshape(_EROWS, _K)
    map2 = mapping.reshape(_EROWS, 16)
    a3, b3 = pl.pallas_call(
        _prep_body,
        out_shape=(
            jax.ShapeDtypeStruct((_CH_PAD, _K), jnp.int32),
            jax.ShapeDtypeStruct((_CH_PAD, _K), jnp.int32),
        ),
    )(f2a2, map2)

    s2 = _sc_edge_sum()(f_atoms, a3, b3)

    mp = jnp.concatenate([
        mapping,
        jnp.full((_MROWS * 128 - _N_GROUPS,), -1, jnp.int32),
    ]).reshape(_MROWS, 128)

    func_save_new, cnt = pl.pallas_call(
        _tc_body,
        out_shape=(
            jax.ShapeDtypeStruct((_N_FUNC, _HIDDEN), jnp.float32),
            jax.ShapeDtypeStruct((_N_FUNC, 1), jnp.int32),
        ),
    )(s2, W_i_atom, func_save, mp)

    return func_save_new, cnt.reshape(_N_FUNC)


# speedup vs baseline: 1.3859x; 1.0070x over previous
import functools

import jax
import jax.numpy as jnp
from jax import lax
from jax.experimental import pallas as pl
from jax.experimental.pallas import tpu as pltpu
from jax.experimental.pallas import tpu_sc as plsc

_N_ATOMS = 100000
_FDIM = 128
_HIDDEN = 300
_N_GROUPS = 20000
_GSIZE = 8
_N_FUNC = 64

_K = 128
_N0 = 72
_N1 = 8
_TOT_CHUNK = 16 * (_N0 + _N1)
_CH_PAD = _TOT_CHUNK + _N0 - _N1
_E_PAD = _CH_PAD * _K
_NBUF = 4
_MROWS = 160
_SLAB_ROWS = 128
_RPT = _N_FUNC // 16


def _sc_body(f_hbm, a_hbm, b_hbm, out_hbm,
             aidx_v, bidx_v, rows_v, acc_v, tmp_v, zv, slabs_sh, gsems):
    cid = lax.axis_index("c")
    sid = lax.axis_index("s")
    base = jnp.where(cid == 0, sid * _N0, 16 * _N0 + sid * _N1)
    nch = jnp.where(cid == 0, _N0, _N1)

    @pl.when(cid == 0)
    def _():
        pltpu.sync_copy(a_hbm.at[pl.ds(base, _N0)], aidx_v)
        pltpu.sync_copy(b_hbm.at[pl.ds(base, _N0)], bidx_v)

    @pl.when(cid == 1)
    def _():
        pltpu.sync_copy(a_hbm.at[pl.ds(base, _N1)], aidx_v.at[pl.ds(0, _N1)])
        pltpu.sync_copy(b_hbm.at[pl.ds(base, _N1)], bidx_v.at[pl.ds(0, _N1)])

    pltpu.async_copy(f_hbm.at[aidx_v.at[0]], rows_v.at[0], gsems.at[0])
    pltpu.async_copy(f_hbm.at[aidx_v.at[1]], rows_v.at[1], gsems.at[1])

    zvec = jnp.zeros((16,), jnp.float32)
    for r in range(8):
        for c in range(_FDIM // 16):
            zv[r, pl.ds(c * 16, 16)] = zvec
    for k in range(9):
        pltpu.sync_copy(zv, slabs_sh.at[sid, pl.ds(k * 8, 8)])

    @pl.loop(0, nch, step=_NBUF)
    def _(j):
        for b in range(_NBUF):
            i = j + b
            pltpu.make_async_copy(f_hbm.at[aidx_v.at[i]], rows_v.at[b],
                                  gsems.at[b]).wait()
            nb = (b + 2) % _NBUF

            @pl.when(i + 2 < nch)
            def _():
                pltpu.async_copy(f_hbm.at[aidx_v.at[i + 2]], rows_v.at[nb],
                                 gsems.at[nb])

            pltpu.sync_copy(rows_v.at[b], slabs_sh.at[sid].at[bidx_v.at[i]],
                            add=True)

    plsc.subcore_barrier()

    r0 = sid * _RPT
    pltpu.sync_copy(slabs_sh.at[0, pl.ds(r0, _RPT)], acc_v)

    @pl.loop(1, 16)
    def _(s):
        pltpu.sync_copy(slabs_sh.at[s, pl.ds(r0, _RPT)], tmp_v)
        for r in range(_RPT):
            for c in range(_FDIM // 16):
                sl = pl.ds(c * 16, 16)
                acc_v[r, sl] = acc_v[r, sl] + tmp_v[r, sl]

    pltpu.sync_copy(acc_v, out_hbm.at[cid, pl.ds(r0, _RPT)])


@functools.cache
def _sc_edge_sum():
    return functools.partial(
        pl.kernel,
        out_type=jax.ShapeDtypeStruct((2, _N_FUNC, _FDIM), jnp.float32),
        mesh=plsc.VectorSubcoreMesh(core_axis_name="c", subcore_axis_name="s"),
        scratch_types=[
            pltpu.VMEM((_N0, _K), jnp.int32),
            pltpu.VMEM((_N0, _K), jnp.int32),
            pltpu.VMEM((_NBUF, _K, _FDIM), jnp.float32),
            pltpu.VMEM((_RPT, _FDIM), jnp.float32),
            pltpu.VMEM((_RPT, _FDIM), jnp.float32),
            pltpu.VMEM((8, _FDIM), jnp.float32),
            pltpu.VMEM_SHARED((16, _SLAB_ROWS, _FDIM), jnp.float32),
            pltpu.SemaphoreType.DMA((_NBUF,)),
        ],
    )(_sc_body)


_EROWS = _N_GROUPS * _GSIZE // _K


def _prep_body(f2a_ref, map_ref, a_ref, b_ref):
    a_blk = f2a_ref[...]
    parts = [jnp.broadcast_to(map_ref[:, pl.ds(g, 1)], (_EROWS, _GSIZE))
             for g in range(16)]
    b_blk = jnp.concatenate(parts, axis=1)
    b_blk = jnp.where(a_blk == 0, _N_FUNC, b_blk)
    a_blk = jnp.maximum(a_blk - 1, 0)
    npad = _CH_PAD - _EROWS
    a_ref[...] = jnp.concatenate(
        [a_blk, jnp.zeros((npad, _K), jnp.int32)], axis=0)
    b_ref[...] = jnp.concatenate(
        [b_blk, jnp.full((npad, _K), _N_FUNC, jnp.int32)], axis=0)


def _tc_body(s2_ref, w_ref, fs_ref, m_ref, out_ref, cnt_ref):
    s = s2_ref[0] + s2_ref[1]
    out_ref[...] = fs_ref[...] + jnp.dot(
        s, w_ref[...], preferred_element_type=jnp.float32)

    iota = lax.broadcasted_iota(jnp.int32, (_N_FUNC, 128), 0)

    def body(r, acc):
        blk = m_ref[pl.ds(r, 1), :]
        return acc + (jnp.broadcast_to(blk, (_N_FUNC, 128)) == iota
                      ).astype(jnp.int32)

    acc = lax.fori_loop(0, _MROWS, body,
                        jnp.zeros((_N_FUNC, 128), jnp.int32))
    cnt_ref[...] = jnp.sum(acc, axis=1, keepdims=True) + 1


def kernel(step, f_atoms, func2atom, mapping, W_i_atom, func_save):
    del step
    mapping = mapping.astype(jnp.int32)
    f2a2 = func2atom.astype(jnp.int32).reshape(_EROWS, _K)
    map2 = mapping.reshape(_EROWS, 16)
    a3, b3 = pl.pallas_call(
        _prep_body,
        out_shape=(
            jax.ShapeDtypeStruct((_CH_PAD, _K), jnp.int32),
            jax.ShapeDtypeStruct((_CH_PAD, _K), jnp.int32),
        ),
    )(f2a2, map2)

    s2 = _sc_edge_sum()(f_atoms, a3, b3)

    mp = jnp.concatenate([
        mapping,
        jnp.full((_MROWS * 128 - _N_GROUPS,), -1, jnp.int32),
    ]).reshape(_MROWS, 128)

    func_save_new, cnt = pl.pallas_call(
        _tc_body,
        out_shape=(
            jax.ShapeDtypeStruct((_N_FUNC, _HIDDEN), jnp.float32),
            jax.ShapeDtypeStruct((_N_FUNC, 1), jnp.int32),
        ),
    )(s2, W_i_atom, func_save, mp)

    return func_save_new, cnt.reshape(_N_FUNC)
